# baseline (speedup 1.0000x reference)
import functools
import jax
import jax.numpy as jnp
from jax import lax
from jax.experimental import pallas as pl
from jax.experimental.pallas import tpu as pltpu
from jax.experimental.pallas import tpu_sc as plsc
import numpy as np

_N = 20000
_NP = 20480
_NR = _NP // 128
_PRE_K = 2000
_KP = 2048
_POST_K = 1000
_QP = 1024
_THR = 0.7
_IMG_H = 1024.0
_IMG_W = 1024.0
_CLAMP = float(np.log(1000.0 / 16.0))
_NEG = -1e9
_MININT = np.int32(-2147483648)
_NW = 32


def _decode_body(d_ref, a_ref, s_ref, bo_ref, so_ref, key_ref, slot_ref):
    f32 = jnp.float32
    i32 = jnp.int32
    dx = d_ref[0]
    dy = d_ref[1]
    dw = jnp.minimum(d_ref[2], _CLAMP)
    dh = jnp.minimum(d_ref[3], _CLAMP)
    a0 = a_ref[0]
    a1 = a_ref[1]
    a2 = a_ref[2]
    a3 = a_ref[3]
    wa = a2 - a0
    ha = a3 - a1
    cxa = a0 + 0.5 * wa
    cya = a1 + 0.5 * ha
    px = dx * wa + cxa
    py = dy * ha + cya
    pw = jnp.exp(dw) * wa
    ph = jnp.exp(dh) * ha
    x1 = jnp.clip(px - 0.5 * pw, 0.0, _IMG_W)
    y1 = jnp.clip(py - 0.5 * ph, 0.0, _IMG_H)
    x2 = jnp.clip(px + 0.5 * pw, 0.0, _IMG_W)
    y2 = jnp.clip(py + 0.5 * ph, 0.0, _IMG_H)
    bo_ref[0] = x1
    bo_ref[1] = y1
    bo_ref[2] = x2
    bo_ref[3] = y2
    valid = ((x2 - x1) > 0.0) & ((y2 - y1) > 0.0)
    s = jnp.where(valid, s_ref[...], _NEG)
    so_ref[...] = s

    b = lax.bitcast_convert_type(s, i32)
    key = jnp.where(b < 0, jnp.bitwise_xor(jnp.invert(b), _MININT), b)
    key_ref[...] = key

    def bit_step(t, acc_u):
        cand_u = jnp.bitwise_or(acc_u, jnp.left_shift(jnp.int32(1), 31 - t))
        cand_s = jnp.bitwise_xor(cand_u, _MININT)
        cnt = jnp.sum((key >= cand_s).astype(f32))
        return jnp.where(cnt >= float(_PRE_K), cand_u, acc_u)

    acc_u = lax.fori_loop(0, 32, bit_step, jnp.int32(0))
    t_s = jnp.bitwise_xor(acc_u, _MININT)

    hi = (key > t_s).astype(f32)
    tie = (key == t_s).astype(f32)
    n_hi = jnp.sum(hi)

    ustrict = (lax.broadcasted_iota(i32, (128, 128), 0) <
               lax.broadcasted_iota(i32, (128, 128), 1)).astype(f32)
    lstrict = (lax.broadcasted_iota(i32, (_NR, _NR), 1) <
               lax.broadcasted_iota(i32, (_NR, _NR), 0)).astype(f32)

    def prefix(flag):
        inrow = lax.dot_general(flag, ustrict, (((1,), (0,)), ((), ())),
                                preferred_element_type=f32,
                                precision=lax.Precision.HIGHEST)
        rowsum = jnp.sum(flag, axis=1, keepdims=True)
        offs = lax.dot_general(lstrict, rowsum, (((1,), (0,)), ((), ())),
                               preferred_element_type=f32,
                               precision=lax.Precision.HIGHEST)
        return inrow + offs

    pos_hi = prefix(hi)
    pos_tie = prefix(tie)
    slot = jnp.where(hi > 0.5, pos_hi,
                     jnp.where(tie > 0.5, n_hi + pos_tie, float(_KP)))
    slot = jnp.minimum(slot, float(_KP))
    slot_ref[...] = slot.astype(i32)


def _decode(d_t, a_t, s_r):
    return pl.pallas_call(
        _decode_body,
        out_shape=[
            jax.ShapeDtypeStruct((4, _NR, 128), jnp.float32),
            jax.ShapeDtypeStruct((_NR, 128), jnp.float32),
            jax.ShapeDtypeStruct((_NR, 128), jnp.int32),
            jax.ShapeDtypeStruct((_NR, 128), jnp.int32),
        ],
    )(d_t, a_t, s_r)


_MESH = plsc.VectorSubcoreMesh(core_axis_name="c", subcore_axis_name="s")


@functools.partial(
    pl.kernel, mesh=_MESH,
    compiler_params=pltpu.CompilerParams(use_tc_tiling_on_sc=False),
    out_type=jax.ShapeDtypeStruct((_KP + 1, 16), jnp.float32),
    scratch_types=[
        pltpu.VMEM((5, 128), jnp.int32),
        pltpu.VMEM((640, 16), jnp.float32),
        pltpu.VMEM((48, 16), jnp.float32),
        pltpu.SemaphoreType.DMA,
    ],
)
def _sc_compact(rows_hbm, idx_hbm, init_hbm, out_hbm, idx_v, rows_v, init_v, sem):
    wid = lax.axis_index("s") * 2 + lax.axis_index("c")

    @pl.when(wid == 0)
    def _():
        pltpu.sync_copy(init_hbm, init_v)
        pltpu.sync_copy(init_v, out_hbm.at[pl.ds(_PRE_K, 48)])

    plsc.subcore_barrier()
    pltpu.sync_copy(rows_hbm.at[wid], rows_v)
    pltpu.sync_copy(idx_hbm.at[wid], idx_v)
    for g in range(5):
        pltpu.async_copy(rows_v.at[pl.ds(g * 128, 128)],
                         out_hbm.at[idx_v.at[g]], sem).wait()


@functools.partial(
    pl.kernel, mesh=_MESH,
    compiler_params=pltpu.CompilerParams(use_tc_tiling_on_sc=False),
    out_type=jax.ShapeDtypeStruct((_KP, 16), jnp.float32),
    scratch_types=[
        pltpu.VMEM((1, 64), jnp.int32),
        pltpu.VMEM((64, 16), jnp.float32),
        pltpu.SemaphoreType.DMA,
    ],
)
def _sc_sort(rows_hbm, idx_hbm, out_hbm, idx_v, rows_v, sem):
    wid = lax.axis_index("s") * 2 + lax.axis_index("c")
    pltpu.sync_copy(rows_hbm.at[wid], rows_v)
    pltpu.sync_copy(idx_hbm.at[wid], idx_v)
    pltpu.async_copy(rows_v, out_hbm.at[idx_v.at[0]], sem).wait()


def _rank_body(krow_ref, kcol_ref, pos_ref):
    f32 = jnp.float32
    i32 = jnp.int32

    def rank_jc(jc, _):
        j0 = jc * 128
        kj = kcol_ref[pl.ds(j0, 128), :]
        jidx = lax.broadcasted_iota(i32, (128, 1), 0) + j0

        def rank_kc(kc, acc):
            k0 = kc * 128
            kk = krow_ref[:, pl.ds(k0, 128)]
            kidx = lax.broadcasted_iota(i32, (1, 128), 1) + k0
            gt = (kk > kj).astype(f32)
            eq = ((kk == kj) & (kidx < jidx)).astype(f32)
            return acc + jnp.sum(gt + eq, axis=1, keepdims=True)

        acc = lax.fori_loop(0, _KP // 128, rank_kc, jnp.zeros((128, 1), f32))
        pos_ref[pl.ds(j0, 128), :] = acc.astype(i32)
        return 0

    lax.fori_loop(0, _KP // 128, rank_jc, 0)


def _rank(krow, kcol):
    return pl.pallas_call(
        _rank_body,
        out_shape=jax.ShapeDtypeStruct((_KP, 1), jnp.int32),
    )(krow, kcol)


def _nms_body(bc_ref, br_ref, ts_ref, tsc_ref, out_ref,
              over_ref, keep_ref, kcol_ref, pos_ref, b_ref, local_ref):
    f32 = jnp.float32
    x1c = bc_ref[0:1, :]
    y1c = bc_ref[1:2, :]
    x2c = bc_ref[2:3, :]
    y2c = bc_ref[3:4, :]
    area_c = (x2c - x1c) * (y2c - y1c)

    def iou_block(b, _):
        r0 = b * 128
        x1r = br_ref[0, pl.ds(r0, 128), :]
        y1r = br_ref[1, pl.ds(r0, 128), :]
        x2r = br_ref[2, pl.ds(r0, 128), :]
        y2r = br_ref[3, pl.ds(r0, 128), :]
        area_r = (x2r - x1r) * (y2r - y1r)
        w = jnp.maximum(jnp.minimum(x2r, x2c) - jnp.maximum(x1r, x1c), 0.0)
        h = jnp.maximum(jnp.minimum(y2r, y2c) - jnp.maximum(y1r, y1c), 0.0)
        inter = w * h
        iou = inter / (area_r + area_c - inter + 1e-9)
        over_ref[pl.ds(r0, 128), :] = (iou > _THR).astype(f32)
        return 0

    lax.fori_loop(0, _KP // 128, iou_block, 0)

    keep_ref[...] = jnp.ones((1, _KP), f32)
    cid = lax.broadcasted_iota(jnp.int32, (1, _KP), 1)
    lid = lax.broadcasted_iota(jnp.int32, (1, 128), 1)
    tri = (lax.broadcasted_iota(jnp.int32, (128, 128), 0) <
           lax.broadcasted_iota(jnp.int32, (128, 128), 1)).astype(f32)

    for b in range(_KP // 128):
        r0 = b * 128
        local_ref[...] = over_ref[r0:r0 + 128, r0:r0 + 128] * tri
        keepb0 = keep_ref[:, r0:r0 + 128]

        def nms_step(i, keepb):
            row = local_ref[pl.ds(i, 1), :]
            ki = jnp.sum(jnp.where(lid == i, keepb, 0.0))
            return keepb * (1.0 - row * ki)

        keepb = lax.fori_loop(0, 128, nms_step, keepb0)
        keep_ref[:, r0:r0 + 128] = keepb
        if b + 1 < _KP // 128:
            counts = lax.dot_general(
                keepb, over_ref[r0:r0 + 128, :],
                (((1,), (0,)), ((), ())),
                preferred_element_type=f32,
                precision=lax.Precision.HIGHEST)
            sup = ((counts > 0.5) & (cid >= r0 + 128)).astype(f32)
            keep_ref[...] = keep_ref[...] * (1.0 - sup)

    keep = keep_ref[...]
    real = cid < _PRE_K
    ks = jnp.where((keep > 0.5) & real, ts_ref[...], _NEG)

    def eye_block(b, _):
        r0 = b * 128
        ri = lax.broadcasted_iota(jnp.int32, (128, _KP), 0) + r0
        ci = lax.broadcasted_iota(jnp.int32, (128, _KP), 1)
        over_ref[pl.ds(r0, 128), :] = (ri == ci).astype(f32)
        return 0

    lax.fori_loop(0, _KP // 128, eye_block, 0)
    eye = over_ref[...]
    kcol_ref[...] = lax.dot_general(
        eye, keep, (((1,), (1,)), ((), ())),
        preferred_element_type=f32,
        precision=lax.Precision.HIGHEST)

    def rank_jc(jc, _):
        j0 = jc * 128
        jidx = lax.broadcasted_iota(jnp.int32, (128, 1), 0) + j0
        kj = jnp.where((kcol_ref[pl.ds(j0, 128), :] > 0.5) & (jidx < _PRE_K),
                       tsc_ref[pl.ds(j0, 128), :], _NEG)

        def rank_kc(kc, acc):
            k0 = kc * 128
            kidx = lax.broadcasted_iota(jnp.int32, (1, 128), 1) + k0
            kk = jnp.where((keep_ref[:, pl.ds(k0, 128)] > 0.5) &
                           (kidx < _PRE_K),
                           ts_ref[:, pl.ds(k0, 128)], _NEG)
            gt = (kk > kj).astype(f32)
            eq = ((kk == kj) & (kidx < jidx)).astype(f32)
            return acc + jnp.sum(gt + eq, axis=1, keepdims=True)

        acc = lax.fori_loop(0, _KP // 128, rank_kc, jnp.zeros((128, 1), f32))
        pos_ref[pl.ds(j0, 128), :] = acc
        return 0

    lax.fori_loop(0, _KP // 128, rank_jc, 0)

    qid = lax.broadcasted_iota(jnp.int32, (1, _QP), 1).astype(f32)

    def onehot_jc(jc, _):
        j0 = jc * 128
        p = pos_ref[pl.ds(j0, 128), :]
        b_ref[pl.ds(j0, 128), :] = (p == qid).astype(f32)
        return 0

    lax.fori_loop(0, _KP // 128, onehot_jc, 0)
    data = jnp.concatenate(
        [bc_ref[...], ks, jnp.zeros((3, _KP), f32)], axis=0)
    out_ref[...] = lax.dot_general(
        data, b_ref[...], (((1,), (0,)), ((), ())),
        preferred_element_type=f32,
        precision=lax.Precision.HIGHEST)


def _nms_order(bc, br, ts_row, ts_col):
    f32 = jnp.float32
    return pl.pallas_call(
        _nms_body,
        out_shape=jax.ShapeDtypeStruct((8, _QP), f32),
        scratch_shapes=[
            pltpu.VMEM((_KP, _KP), f32),
            pltpu.VMEM((1, _KP), f32),
            pltpu.VMEM((_KP, 1), f32),
            pltpu.VMEM((_KP, 1), f32),
            pltpu.VMEM((_KP, _QP), f32),
            pltpu.VMEM((128, 128), f32),
        ],
    )(bc, br, ts_row, ts_col)


@jax.jit
def kernel(scores, deltas, anchors):
    f32 = jnp.float32
    i32 = jnp.int32
    pad = _NP - _N
    s_p = jnp.pad(scores, (0, pad))
    d_t = jnp.pad(deltas, ((0, pad), (0, 0))).T.reshape(4, _NR, 128)
    a_t = jnp.pad(anchors, ((0, pad), (0, 0))).T.reshape(4, _NR, 128)
    s_r = s_p.reshape(_NR, 128)

    boxes, s_m, key, slot = _decode(d_t, a_t, s_r)

    keyf = lax.bitcast_convert_type(key, f32).reshape(_NP, 1)
    rows = jnp.concatenate(
        [boxes.reshape(4, _NP).T, s_m.reshape(_NP, 1), keyf,
         jnp.zeros((_NP, 10), f32)], axis=1)
    init = jnp.concatenate(
        [jnp.zeros((48, 5), f32),
         jnp.broadcast_to(lax.bitcast_convert_type(_MININT, f32), (48, 1)),
         jnp.zeros((48, 10), f32)], axis=1)

    comp = _sc_compact(rows.reshape(_NW, _NP // _NW, 16),
                       slot.reshape(_NW, 5, 128), init)[:_KP]

    ckey = lax.bitcast_convert_type(comp[:, 5], i32)
    ranks = _rank(ckey.reshape(1, _KP), ckey.reshape(_KP, 1))

    srt = _sc_sort(comp.reshape(_NW, _KP // _NW, 16),
                   ranks.reshape(_NW, 1, _KP // _NW))

    bc = srt[:, :4].T
    ts = srt[:, 4]
    br = bc.reshape(4, 1, _KP).transpose(0, 2, 1)
    out_t = _nms_order(bc, br, ts.reshape(1, _KP), ts.reshape(_KP, 1))
    return out_t[:5, :_POST_K].T
